# baseline (speedup 1.0000x reference)
import jax
import jax.numpy as jnp
from jax import lax
from jax.experimental import pallas as pl
from jax.experimental.pallas import tpu as pltpu
from jax.experimental.pallas import tpu_sc as plsc

_N = 1000000
_B = 16384
_F = 64
_NW = 32
_BPW = _B // _NW
_G = 16
_GROUPS = _BPW // _G
_NBLK = (_N + 127) // 128
_BPR = (_NBLK + _NW - 1) // _NW
_NCH = _B // 16
_BMCH = 16
_RING = 16
_DEPTH = 6


def _gather_body(user_hbm, item_hbm, utab_hbm, itab_hbm, euf_hbm, eif_hbm,
                 allidx_v, cand_v, mb_v, mc_v, bitmap_v, hb_v,
                 blk_v, stage_v, sem, semw):
    nc = 2
    wid = lax.axis_index("s") * nc + lax.axis_index("c")
    lane = lax.iota(jnp.int32, 16)
    lo = wid * _BPR
    hi = jnp.minimum(lo + _BPR, _NBLK)
    nloc = hi - lo
    zeros16 = jnp.zeros((16,), jnp.int32)
    ones16 = jnp.full((16,), 1, jnp.int32)

    now = zeros16

    for idx_hbm, tab_hbm, out_hbm in ((user_hbm, utab_hbm, euf_hbm),
                                      (item_hbm, itab_hbm, eif_hbm)):
        pltpu.sync_copy(idx_hbm, allidx_v)
        for t in range(_BMCH):
            bitmap_v[pl.ds(t * 16, 16)] = zeros16

        def cand_body(k, base):
            idxv = allidx_v[pl.ds(k * 16, 16)]
            bvals = k * 16 + lane
            bid = lax.shift_right_logical(idxv, 7)
            m = jnp.logical_and(bid >= lo, bid < hi)
            mi = m.astype(jnp.int32)
            pos = base + plsc.cumsum(mi) - 1
            plsc.store_scatter(cand_v, [pos], bvals, mask=m)
            plsc.store_scatter(bitmap_v, [bid - lo], ones16, mask=m)
            return base + plsc.all_reduce_population_count(m)

        base = lax.fori_loop(0, _NCH, cand_body, zeros16, unroll=False)
        ncand = base[0]

        def hb_body(t, hbase):
            bm = bitmap_v[pl.ds(t * 16, 16)]
            loc = t * 16 + lane
            m = jnp.logical_and(bm > 0, loc < nloc)
            mi = m.astype(jnp.int32)
            pos = hbase + plsc.cumsum(mi) - 1
            plsc.store_scatter(hb_v, [pos], loc + lo, mask=m)
            return hbase + plsc.all_reduce_population_count(m)

        hbase = lax.fori_loop(0, _BMCH, hb_body, zeros16, unroll=False)
        nhb = hbase[0]
        ncchunk = lax.div(ncand + 15, 16)

        def fetch(t, buf):
            tv = jnp.full((16,), t, jnp.int32)
            blk = plsc.load_gather(hb_v, [tv])[0]
            off = pl.multiple_of(blk * 128, 128)
            pltpu.async_copy(tab_hbm.at[:, pl.ds(off, 128)],
                             blk_v.at[buf], sem)

        def drain(buf):
            pltpu.make_async_copy(tab_hbm.at[:, pl.ds(0, 128)],
                                  blk_v.at[buf], sem).wait()

        def process(t, buf, nw):
            tv = jnp.full((16,), t, jnp.int32)
            blkv = plsc.load_gather(hb_v, [tv])

            def chunk_body(c, nw_c):
                valid = c * 16 + lane < ncand
                cb = plsc.load_gather(cand_v, [c * 16 + lane], mask=valid)
                civ = plsc.load_gather(allidx_v, [cb], mask=valid)
                m = jnp.logical_and(valid,
                                    lax.shift_right_logical(civ, 7) == blkv)
                mi = m.astype(jnp.int32)
                pos = plsc.cumsum(mi) - 1
                plsc.store_scatter(mb_v, [pos], cb, mask=m)
                plsc.store_scatter(mc_v, [pos], jnp.bitwise_and(civ, 127),
                                   mask=m)
                cnt = plsc.all_reduce_population_count(m)

                def match_body(j, carry2):
                    ordv = nw_c + j
                    slot = lax.rem(ordv[0], _RING)

                    @pl.when(ordv[0] >= _RING)
                    def _():
                        pltpu.make_async_copy(
                            euf_hbm.at[pl.ds(0, _F)],
                            stage_v.at[0], semw).wait()

                    jv = jnp.full((16,), j, jnp.int32)
                    bsc = plsc.load_gather(mb_v, [jv])[0]
                    cmod = plsc.load_gather(mc_v, [jv])
                    for k in range(_F // 16):
                        stage_v[slot, pl.ds(k * 16, 16)] = plsc.load_gather(
                            blk_v.at[buf], [k * 16 + lane, cmod])
                    pltpu.async_copy(stage_v.at[slot],
                                     out_hbm.at[pl.ds(bsc * _F, _F)], semw)
                    return carry2

                lax.fori_loop(0, cnt[0], match_body, 0, unroll=False)
                return nw_c + cnt

            return lax.fori_loop(0, ncchunk, chunk_body, nw, unroll=False)

        for p0 in range(_DEPTH - 1):
            @pl.when(p0 < nhb)
            def _(p0=p0):
                fetch(p0, p0)

        def blk_body(t, nw):
            buf = lax.rem(t, _DEPTH)

            @pl.when(t + _DEPTH - 1 < nhb)
            def _():
                fetch(t + _DEPTH - 1, lax.rem(t + _DEPTH - 1, _DEPTH))

            drain(buf)
            return process(t, buf, nw)

        now = lax.fori_loop(0, nhb, blk_body, now, unroll=False)

    def drainw_body(j, carry):
        pltpu.make_async_copy(euf_hbm.at[pl.ds(0, _F)],
                              stage_v.at[0], semw).wait()
        return carry

    lax.fori_loop(0, jnp.minimum(now[0], _RING), drainw_body, 0,
                  unroll=False)


def _combine_body(euf_hbm, eif_hbm, wb_hbm, out_hbm,
                  eu_v, ei_v, wb_v, out_v, sem):
    nc = 2
    wid = lax.axis_index("s") * nc + lax.axis_index("c")
    base = wid * _BPW

    pltpu.sync_copy(euf_hbm.at[pl.ds(base * _F, _BPW * _F)], eu_v)
    pltpu.sync_copy(eif_hbm.at[pl.ds(base * _F, _BPW * _F)], ei_v)
    pltpu.sync_copy(wb_hbm, wb_v)

    wvecs = [wb_v[pl.ds(c * 16, 16)] for c in range(_F // 16)]
    bvec = wb_v[pl.ds(_F, 16)]
    lane = lax.iota(jnp.int32, 16)

    def group_body(g, carry):
        goff = g * _G
        rows = goff + lane
        acc = bvec
        for f in range(_F):
            wf = wvecs[f // 16][f % 16]
            flat = rows * _F + f
            u = plsc.load_gather(eu_v, [flat])
            iv = plsc.load_gather(ei_v, [flat])
            acc = acc + u * iv * wf
        out_v[pl.ds(goff, _G)] = acc
        return carry

    lax.fori_loop(0, _GROUPS, group_body, 0, unroll=False)
    pltpu.sync_copy(out_v, out_hbm.at[pl.ds(base, _BPW)])


def kernel(user, item, embed_user_w, embed_item_w, predict_w, predict_b):
    wb = jnp.concatenate([predict_w.reshape(_F).astype(jnp.float32),
                          jnp.broadcast_to(predict_b.astype(jnp.float32), (16,))])
    mesh = plsc.VectorSubcoreMesh(core_axis_name="c", subcore_axis_name="s")

    run_gather = pl.kernel(
        _gather_body,
        out_type=(jax.ShapeDtypeStruct((_B * _F,), jnp.float32),
                  jax.ShapeDtypeStruct((_B * _F,), jnp.float32)),
        mesh=mesh,
        compiler_params=pltpu.CompilerParams(needs_layout_passes=False,
                                             use_tc_tiling_on_sc=True),
        scratch_types=[
            pltpu.VMEM((_B,), jnp.int32),
            pltpu.VMEM((_B,), jnp.int32),
            pltpu.VMEM((16,), jnp.int32),
            pltpu.VMEM((16,), jnp.int32),
            pltpu.VMEM((_BMCH * 16,), jnp.int32),
            pltpu.VMEM((_BMCH * 16,), jnp.int32),
            pltpu.VMEM((_DEPTH, _F, 128), jnp.float32),
            pltpu.VMEM((_RING, _F), jnp.float32),
            pltpu.SemaphoreType.DMA,
            pltpu.SemaphoreType.DMA,
        ],
    )
    euf, eif = run_gather(user.astype(jnp.int32), item.astype(jnp.int32),
                          embed_user_w.T, embed_item_w.T)

    run_combine = pl.kernel(
        _combine_body,
        out_type=jax.ShapeDtypeStruct((_B,), jnp.float32),
        mesh=mesh,
        compiler_params=pltpu.CompilerParams(needs_layout_passes=False,
                                             use_tc_tiling_on_sc=False),
        scratch_types=[
            pltpu.VMEM((_BPW * _F,), jnp.float32),
            pltpu.VMEM((_BPW * _F,), jnp.float32),
            pltpu.VMEM((_F + 16,), jnp.float32),
            pltpu.VMEM((_BPW,), jnp.float32),
            pltpu.SemaphoreType.DMA,
        ],
    )
    return run_combine(euf, eif, wb)


# speedup vs baseline: 2.6767x; 2.6767x over previous
import jax
import jax.numpy as jnp
from jax import lax
from jax.experimental import pallas as pl
from jax.experimental.pallas import tpu as pltpu
from jax.experimental.pallas import tpu_sc as plsc

_N = 1000000
_B = 16384
_F = 64
_NW = 32
_BPW = _B // _NW
_G = 16
_GROUPS = _BPW // _G
_NBLK = (_N + 127) // 128
_BPR = (_NBLK + _NW - 1) // _NW
_NCH = _B // 16
_BMCH = 16
_RING = 16
_DEPTH = 6


def _gather_body(user_hbm, item_hbm, utab_hbm, itab_hbm, euf_hbm, eif_hbm,
                 allidx_v, cand_v, mb_v, mc_v, bitmap_v, hb_v,
                 csort_v, cnt_v, starts_v, tmp_v,
                 blk_v, stage_v, sem, semw):
    nc = 2
    wid = lax.axis_index("s") * nc + lax.axis_index("c")
    lane = lax.iota(jnp.int32, 16)
    lo = wid * _BPR
    hi = jnp.minimum(lo + _BPR, _NBLK)
    nloc = hi - lo
    zeros16 = jnp.zeros((16,), jnp.int32)
    ones16 = jnp.full((16,), 1, jnp.int32)

    now = zeros16

    for idx_hbm, tab_hbm, out_hbm in ((user_hbm, utab_hbm, euf_hbm),
                                      (item_hbm, itab_hbm, eif_hbm)):
        pltpu.sync_copy(idx_hbm, allidx_v)
        for t in range(_BMCH):
            bitmap_v[pl.ds(t * 16, 16)] = zeros16

        def cand_body(k, base):
            idxv = allidx_v[pl.ds(k * 16, 16)]
            bvals = k * 16 + lane
            bid = lax.shift_right_logical(idxv, 7)
            m = jnp.logical_and(bid >= lo, bid < hi)
            mi = m.astype(jnp.int32)
            pos = base + plsc.cumsum(mi) - 1
            plsc.store_scatter(cand_v, [pos], bvals, mask=m)
            plsc.store_scatter(bitmap_v, [bid - lo], ones16, mask=m)
            return base + plsc.all_reduce_population_count(m)

        base = lax.fori_loop(0, _NCH, cand_body, zeros16, unroll=False)
        ncand = base[0]

        def hb_body(t, hbase):
            bm = bitmap_v[pl.ds(t * 16, 16)]
            loc = t * 16 + lane
            m = jnp.logical_and(bm > 0, loc < nloc)
            mi = m.astype(jnp.int32)
            pos = hbase + plsc.cumsum(mi) - 1
            plsc.store_scatter(hb_v, [pos], loc + lo, mask=m)
            return hbase + plsc.all_reduce_population_count(m)

        hbase = lax.fori_loop(0, _BMCH, hb_body, zeros16, unroll=False)
        nhb = hbase[0]
        ncchunk = lax.div(ncand + 15, 16)

        for t in range(_BMCH):
            cnt_v[pl.ds(t * 16, 16)] = zeros16

        def cntA_body(c, carry):
            valid = c * 16 + lane < ncand
            cb = plsc.load_gather(cand_v, [c * 16 + lane], mask=valid)
            civ = plsc.load_gather(allidx_v, [cb], mask=valid)
            bidl = lax.shift_right_logical(civ, 7) - lo
            plsc.addupdate_scatter(cnt_v, [bidl], ones16, mask=valid)
            return carry

        lax.fori_loop(0, ncchunk, cntA_body, 0, unroll=False)

        carry0 = zeros16
        fill_slots = []
        for t in range(_BMCH):
            ch = cnt_v[pl.ds(t * 16, 16)]
            cs = plsc.cumsum(ch)
            ex = carry0 + cs - ch
            starts_v[pl.ds(t * 16, 16)] = ex
            fill_slots.append((t, ex))
            carry0 = carry0 + jnp.full((16,), 1, jnp.int32) * cs[15]
        for t, ex in fill_slots:
            bitmap_v[pl.ds(t * 16, 16)] = ex

        def place_body(c, carry):
            valid = c * 16 + lane < ncand
            cb = plsc.load_gather(cand_v, [c * 16 + lane], mask=valid)
            civ = plsc.load_gather(allidx_v, [cb], mask=valid)
            bidl = lax.shift_right_logical(civ, 7) - lo
            bkey = jnp.where(valid, bidl, 30000)
            skey, sval = plsc.sort_key_val(bkey, cb)
            tmp_v[pl.ds(0, 16)] = skey
            prev = plsc.load_gather(tmp_v, [jnp.maximum(lane - 1, 0)])
            isstart = jnp.logical_or(lane == 0, skey != prev)
            runstart = plsc.cummax(jnp.where(isstart, lane, 0))
            rank = lane - runstart
            vs = skey != 30000
            base = plsc.load_gather(bitmap_v, [skey], mask=vs)
            plsc.addupdate_scatter(bitmap_v, [skey], ones16, mask=vs)
            plsc.store_scatter(csort_v, [base + rank], sval, mask=vs)
            return carry

        lax.fori_loop(0, ncchunk, place_body, 0, unroll=False)

        def fetch(t, buf):
            tv = jnp.full((16,), t, jnp.int32)
            blk = plsc.load_gather(hb_v, [tv])[0]
            off = pl.multiple_of(blk * 128, 128)
            pltpu.async_copy(tab_hbm.at[:, pl.ds(off, 128)],
                             blk_v.at[buf], sem)

        def drain(buf):
            pltpu.make_async_copy(tab_hbm.at[:, pl.ds(0, 128)],
                                  blk_v.at[buf], sem).wait()

        def process(t, buf, nw):
            tv = jnp.full((16,), t, jnp.int32)
            blkv = plsc.load_gather(hb_v, [tv]) - lo
            st = plsc.load_gather(starts_v, [blkv])[0]
            ct = plsc.load_gather(cnt_v, [blkv])[0]

            def chunk_body(c, nw_c):
                valid = c * 16 + lane < ct
                cb = plsc.load_gather(csort_v, [st + c * 16 + lane],
                                      mask=valid)
                civ = plsc.load_gather(allidx_v, [cb], mask=valid)
                mb_v[pl.ds(0, 16)] = cb
                mc_v[pl.ds(0, 16)] = jnp.bitwise_and(civ, 127)
                cnt = jnp.minimum(jnp.full((16,), 16, jnp.int32),
                                  ct - c * 16)

                def match_body(j, carry2):
                    ordv = nw_c + j
                    slot = lax.rem(ordv[0], _RING)

                    @pl.when(ordv[0] >= _RING)
                    def _():
                        pltpu.make_async_copy(
                            euf_hbm.at[pl.ds(0, _F)],
                            stage_v.at[0], semw).wait()

                    jv = jnp.full((16,), j, jnp.int32)
                    bsc = plsc.load_gather(mb_v, [jv])[0]
                    cmod = plsc.load_gather(mc_v, [jv])
                    for k in range(_F // 16):
                        stage_v[slot, pl.ds(k * 16, 16)] = plsc.load_gather(
                            blk_v.at[buf], [k * 16 + lane, cmod])
                    pltpu.async_copy(stage_v.at[slot],
                                     out_hbm.at[pl.ds(bsc * _F, _F)], semw)
                    return carry2

                lax.fori_loop(0, cnt[0], match_body, 0, unroll=False)
                return nw_c + cnt

            return lax.fori_loop(0, lax.div(ct + 15, 16), chunk_body, nw,
                                 unroll=False)

        for p0 in range(_DEPTH - 1):
            @pl.when(p0 < nhb)
            def _(p0=p0):
                fetch(p0, p0)

        def blk_body(t, nw):
            buf = lax.rem(t, _DEPTH)

            @pl.when(t + _DEPTH - 1 < nhb)
            def _():
                fetch(t + _DEPTH - 1, lax.rem(t + _DEPTH - 1, _DEPTH))

            drain(buf)
            return process(t, buf, nw)

        now = lax.fori_loop(0, nhb, blk_body, now, unroll=False)

    def drainw_body(j, carry):
        pltpu.make_async_copy(euf_hbm.at[pl.ds(0, _F)],
                              stage_v.at[0], semw).wait()
        return carry

    lax.fori_loop(0, jnp.minimum(now[0], _RING), drainw_body, 0,
                  unroll=False)


def _combine_body(euf_hbm, eif_hbm, wb_hbm, out_hbm,
                  eu_v, ei_v, wb_v, out_v, sem):
    nc = 2
    wid = lax.axis_index("s") * nc + lax.axis_index("c")
    base = wid * _BPW

    pltpu.sync_copy(euf_hbm.at[pl.ds(base * _F, _BPW * _F)], eu_v)
    pltpu.sync_copy(eif_hbm.at[pl.ds(base * _F, _BPW * _F)], ei_v)
    pltpu.sync_copy(wb_hbm, wb_v)

    wvecs = [wb_v[pl.ds(c * 16, 16)] for c in range(_F // 16)]
    bvec = wb_v[pl.ds(_F, 16)]
    lane = lax.iota(jnp.int32, 16)

    def group_body(g, carry):
        goff = g * _G
        rows = goff + lane
        acc = bvec
        for f in range(_F):
            wf = wvecs[f // 16][f % 16]
            flat = rows * _F + f
            u = plsc.load_gather(eu_v, [flat])
            iv = plsc.load_gather(ei_v, [flat])
            acc = acc + u * iv * wf
        out_v[pl.ds(goff, _G)] = acc
        return carry

    lax.fori_loop(0, _GROUPS, group_body, 0, unroll=False)
    pltpu.sync_copy(out_v, out_hbm.at[pl.ds(base, _BPW)])


def kernel(user, item, embed_user_w, embed_item_w, predict_w, predict_b):
    wb = jnp.concatenate([predict_w.reshape(_F).astype(jnp.float32),
                          jnp.broadcast_to(predict_b.astype(jnp.float32), (16,))])
    mesh = plsc.VectorSubcoreMesh(core_axis_name="c", subcore_axis_name="s")

    run_gather = pl.kernel(
        _gather_body,
        out_type=(jax.ShapeDtypeStruct((_B * _F,), jnp.float32),
                  jax.ShapeDtypeStruct((_B * _F,), jnp.float32)),
        mesh=mesh,
        compiler_params=pltpu.CompilerParams(needs_layout_passes=False,
                                             use_tc_tiling_on_sc=True),
        scratch_types=[
            pltpu.VMEM((_B,), jnp.int32),
            pltpu.VMEM((_B,), jnp.int32),
            pltpu.VMEM((16,), jnp.int32),
            pltpu.VMEM((16,), jnp.int32),
            pltpu.VMEM((_BMCH * 16,), jnp.int32),
            pltpu.VMEM((_BMCH * 16,), jnp.int32),
            pltpu.VMEM((_B,), jnp.int32),
            pltpu.VMEM((_BMCH * 16,), jnp.int32),
            pltpu.VMEM((_BMCH * 16,), jnp.int32),
            pltpu.VMEM((16,), jnp.int32),
            pltpu.VMEM((_DEPTH, _F, 128), jnp.float32),
            pltpu.VMEM((_RING, _F), jnp.float32),
            pltpu.SemaphoreType.DMA,
            pltpu.SemaphoreType.DMA,
        ],
    )
    euf, eif = run_gather(user.astype(jnp.int32), item.astype(jnp.int32),
                          embed_user_w.T, embed_item_w.T)

    run_combine = pl.kernel(
        _combine_body,
        out_type=jax.ShapeDtypeStruct((_B,), jnp.float32),
        mesh=mesh,
        compiler_params=pltpu.CompilerParams(needs_layout_passes=False,
                                             use_tc_tiling_on_sc=False),
        scratch_types=[
            pltpu.VMEM((_BPW * _F,), jnp.float32),
            pltpu.VMEM((_BPW * _F,), jnp.float32),
            pltpu.VMEM((_F + 16,), jnp.float32),
            pltpu.VMEM((_BPW,), jnp.float32),
            pltpu.SemaphoreType.DMA,
        ],
    )
    return run_combine(euf, eif, wb)
